# baseline (speedup 1.0000x reference)
import functools

import jax
import jax.numpy as jnp
from jax import lax
from jax.experimental import pallas as pl
from jax.experimental.pallas import tpu as pltpu
from jax.experimental.pallas import tpu_sc as plsc

VOCAB = 21128
H = 768
MAX_POS = 512
B = 128
S = 512
EPS = 1e-12

L = 16
NW = 32
SCHUNK = S // NW
KV = H // L
WTOK = B * SCHUNK

_mesh = plsc.VectorSubcoreMesh(core_axis_name="c", subcore_axis_name="s")


@functools.partial(
    pl.kernel,
    out_type=jax.ShapeDtypeStruct((B, S // 8, H // 128, 8, 128),
                                  jnp.float32),
    mesh=_mesh,
    compiler_params=pltpu.CompilerParams(use_tc_tiling_on_sc=False,
                                         needs_layout_passes=False),
    scratch_types=[
        pltpu.VMEM((B, SCHUNK), jnp.int32),
        pltpu.VMEM((B, SCHUNK), jnp.int32),
        pltpu.VMEM((2 * SCHUNK, H), jnp.float32),
        pltpu.VMEM((2, H), jnp.float32),
        pltpu.VMEM((H,), jnp.float32),
        pltpu.VMEM((H,), jnp.float32),
        pltpu.VMEM((SCHUNK, H), jnp.float32),
        pltpu.VMEM((SCHUNK, H), jnp.float32),
        pltpu.VMEM((SCHUNK // 8, H // 128, 8, 128), jnp.float32),
        pltpu.VMEM((SCHUNK // 8, H // 128, 8, 128), jnp.float32),
        pltpu.VMEM((SCHUNK, H), jnp.float32),
        pltpu.VMEM((SCHUNK, L), jnp.float32),
        pltpu.VMEM((SCHUNK, L), jnp.float32),
        pltpu.VMEM((2 * H,), jnp.bfloat16),
        pltpu.SemaphoreType.DMA,
        pltpu.SemaphoreType.DMA,
        pltpu.SemaphoreType.DMA,
        pltpu.SemaphoreType.DMA,
    ],
)
def _emb_ln_kernel(ids_hbm, tt_hbm, w_hbm, pos_hbm, ttemb_hbm, g_hbm, bb_hbm,
                   out_hbm, ids_v, ttv, base_v, ttab_v, g_v, b_v, rows_a,
                   rows_b, ob_a, ob_b, xbuf_v, accs_v, accq_v, gb_v,
                   gsem_a, gsem_b, osem_a, osem_b):
    wid = lax.axis_index("s") * 2 + lax.axis_index("c")
    s0 = wid * SCHUNK

    pltpu.sync_copy(ids_hbm.at[:, pl.ds(s0, SCHUNK)], ids_v)
    pltpu.sync_copy(tt_hbm.at[:, pl.ds(s0, SCHUNK)], ttv)
    pltpu.sync_copy(pos_hbm.at[pl.ds(s0, SCHUNK)], base_v.at[pl.ds(0, SCHUNK)])
    pltpu.sync_copy(pos_hbm.at[pl.ds(s0, SCHUNK)],
                    base_v.at[pl.ds(SCHUNK, SCHUNK)])
    pltpu.sync_copy(ttemb_hbm, ttab_v)
    pltpu.sync_copy(g_hbm, g_v)
    pltpu.sync_copy(bb_hbm, b_v)

    def _mkbase(j, carry):
        for k in range(KV):
            sl = pl.ds(k * L, L)
            base_v[j, sl] = base_v[j, sl] + ttab_v[0, sl]
            base_v[SCHUNK + j, sl] = base_v[SCHUNK + j, sl] + ttab_v[1, sl]
        return carry

    lax.fori_loop(0, SCHUNK, _mkbase, 0)

    for k in range(KV):
        sl = pl.ds(k * L, L)
        gb_v[pl.ds(k * 2 * L, 2 * L)] = plsc.pack(
            g_v[sl], b_v[sl], format=plsc.PackFormat.INTERLEAVED)

    inv_h = jnp.float32(1.0 / H)
    lane0 = jnp.arange(L, dtype=jnp.int32)

    def _start_gather(b, rows_v, gsem):
        idx = ids_v[b, :]
        pltpu.make_async_copy(w_hbm.at[idx], rows_v, gsem).start()

    def _compute(b, rows_v, ob_v):
        tv = ttv[b, :]

        def _ja(j, c2):
            tsp = jnp.take_along_axis(tv, jnp.full((L,), j, jnp.int32),
                                      axis=0)
            r = j + tsp[0] * SCHUNK
            acc_s = [jnp.zeros((L,), jnp.float32) for _ in range(4)]
            acc_q = [jnp.zeros((L,), jnp.float32) for _ in range(4)]
            for k in range(KV):
                sl = pl.ds(k * L, L)
                x = rows_v[j, sl] + base_v[r, sl]
                xbuf_v[j, sl] = x
                acc_s[k % 4] = acc_s[k % 4] + x
                acc_q[k % 4] = acc_q[k % 4] + x * x
            accs_v[j, :] = (acc_s[0] + acc_s[1]) + (acc_s[2] + acc_s[3])
            accq_v[j, :] = (acc_q[0] + acc_q[1]) + (acc_q[2] + acc_q[3])
            return c2

        lax.fori_loop(0, SCHUNK, _ja, 0)

        s4 = [jnp.zeros((L,), jnp.float32) for _ in range(4)]
        q4 = [jnp.zeros((L,), jnp.float32) for _ in range(4)]
        for l in range(L):
            cl = jnp.full((L,), l, jnp.int32)
            s4[l % 4] = s4[l % 4] + plsc.load_gather(accs_v, [lane0, cl])
            q4[l % 4] = q4[l % 4] + plsc.load_gather(accq_v, [lane0, cl])
        s_tot = (s4[0] + s4[1]) + (s4[2] + s4[3])
        q_tot = (q4[0] + q4[1]) + (q4[2] + q4[3])
        mean = s_tot * inv_h
        var = q_tot * inv_h - mean * mean
        vs = var + EPS
        iv = lax.bitcast_convert_type(vs, jnp.int32)
        y = lax.bitcast_convert_type(
            jnp.full((L,), 0x5F3759DF, jnp.int32) - (iv >> 1), jnp.float32)
        for _ in range(3):
            y = y * (1.5 - 0.5 * vs * y * y)
        cvec = mean * y
        ispl = [jnp.take_along_axis(y, jnp.full((L,), j, jnp.int32), axis=0)
                for j in range(SCHUNK)]
        cspl = [jnp.take_along_axis(cvec, jnp.full((L,), j, jnp.int32),
                                    axis=0)
                for j in range(SCHUNK)]

        def _kc(kt, c2):
            for k8 in range(8):
                slk = pl.ds(kt * 128 + k8 * L, L)
                gk, bk = plsc.unpack(gb_v[pl.ds(kt * 256 + k8 * 2 * L, 2 * L)],
                                     format=plsc.PackFormat.INTERLEAVED)
                co = pl.ds(k8 * L, L)
                for j in range(SCHUNK):
                    t = xbuf_v[j, slk] * ispl[j] - cspl[j]
                    ob_v[j // 8, kt, j % 8, co] = t * gk + bk
            return c2

        lax.fori_loop(0, H // 128, _kc, 0)

    idx0 = ids_v[0, :]

    def _phase(b, i, rows_v, gsem, ob_v, osem):
        pltpu.make_async_copy(w_hbm.at[idx0], rows_v, gsem).wait()

        @pl.when(i > 0)
        def _():
            pltpu.make_async_copy(ob_v,
                                  out_hbm.at[b, pl.ds(2 * wid, SCHUNK // 8)],
                                  osem).wait()

        _compute(b, rows_v, ob_v)
        pltpu.make_async_copy(ob_v,
                              out_hbm.at[b, pl.ds(2 * wid, SCHUNK // 8)],
                              osem).start()
        _start_gather(jnp.minimum(b + 2, B - 1), rows_v, gsem)

    _start_gather(0, rows_a, gsem_a)
    _start_gather(1, rows_b, gsem_b)

    def _pair(i, carry):
        b0 = 2 * i
        _phase(b0, i, rows_a, gsem_a, ob_a, osem_a)
        _phase(b0 + 1, i, rows_b, gsem_b, ob_b, osem_b)
        return carry

    lax.fori_loop(0, B // 2, _pair, 0)
    pltpu.make_async_copy(ob_a,
                          out_hbm.at[B - 2, pl.ds(2 * wid, SCHUNK // 8)],
                          osem_a).wait()
    pltpu.make_async_copy(ob_b,
                          out_hbm.at[B - 1, pl.ds(2 * wid, SCHUNK // 8)],
                          osem_b).wait()
    pltpu.make_async_copy(w_hbm.at[idx0], rows_a, gsem_a).wait()
    pltpu.make_async_copy(w_hbm.at[idx0], rows_b, gsem_b).wait()


def kernel(input_ids, token_type_ids, word_embeddings, position_embeddings,
           token_type_embeddings, ln_weight, ln_bias):
    out5 = _emb_ln_kernel(input_ids.astype(jnp.int32),
                          token_type_ids.astype(jnp.int32),
                          word_embeddings, position_embeddings,
                          token_type_embeddings, ln_weight, ln_bias)
    return out5.transpose(0, 1, 3, 2, 4).reshape(B, S, H)


# speedup vs baseline: 1.0007x; 1.0007x over previous
import functools

import jax
import jax.numpy as jnp
from jax import lax
from jax.experimental import pallas as pl
from jax.experimental.pallas import tpu as pltpu
from jax.experimental.pallas import tpu_sc as plsc

VOCAB = 21128
H = 768
MAX_POS = 512
B = 128
S = 512
EPS = 1e-12

L = 16
NW = 32
SCHUNK = S // NW
KV = H // L
WTOK = B * SCHUNK

_mesh = plsc.VectorSubcoreMesh(core_axis_name="c", subcore_axis_name="s")


@functools.partial(
    pl.kernel,
    out_type=jax.ShapeDtypeStruct((B, S * H), jnp.float32),
    mesh=_mesh,
    compiler_params=pltpu.CompilerParams(use_tc_tiling_on_sc=False,
                                         needs_layout_passes=False),
    scratch_types=[
        pltpu.VMEM((B, SCHUNK), jnp.int32),
        pltpu.VMEM((B, SCHUNK), jnp.int32),
        pltpu.VMEM((2 * SCHUNK, H), jnp.float32),
        pltpu.VMEM((2, H), jnp.float32),
        pltpu.VMEM((H,), jnp.float32),
        pltpu.VMEM((H,), jnp.float32),
        pltpu.VMEM((SCHUNK, H), jnp.float32),
        pltpu.VMEM((SCHUNK, H), jnp.float32),
        pltpu.VMEM((SCHUNK * H,), jnp.float32),
        pltpu.VMEM((SCHUNK * H,), jnp.float32),
        pltpu.VMEM((SCHUNK, H), jnp.float32),
        pltpu.VMEM((SCHUNK, L), jnp.float32),
        pltpu.VMEM((SCHUNK, L), jnp.float32),
        pltpu.VMEM((2 * H,), jnp.bfloat16),
        pltpu.SemaphoreType.DMA,
        pltpu.SemaphoreType.DMA,
        pltpu.SemaphoreType.DMA,
        pltpu.SemaphoreType.DMA,
    ],
)
def _emb_ln_kernel(ids_hbm, tt_hbm, w_hbm, pos_hbm, ttemb_hbm, g_hbm, bb_hbm,
                   out_hbm, ids_v, ttv, base_v, ttab_v, g_v, b_v, rows_a,
                   rows_b, ob_a, ob_b, xbuf_v, accs_v, accq_v, gb_v,
                   gsem_a, gsem_b, osem_a, osem_b):
    wid = lax.axis_index("s") * 2 + lax.axis_index("c")
    s0 = wid * SCHUNK

    pltpu.sync_copy(ids_hbm.at[:, pl.ds(s0, SCHUNK)], ids_v)
    pltpu.sync_copy(tt_hbm.at[:, pl.ds(s0, SCHUNK)], ttv)
    pltpu.sync_copy(pos_hbm.at[pl.ds(s0, SCHUNK)], base_v.at[pl.ds(0, SCHUNK)])
    pltpu.sync_copy(pos_hbm.at[pl.ds(s0, SCHUNK)],
                    base_v.at[pl.ds(SCHUNK, SCHUNK)])
    pltpu.sync_copy(ttemb_hbm, ttab_v)
    pltpu.sync_copy(g_hbm, g_v)
    pltpu.sync_copy(bb_hbm, b_v)

    def _mkbase(j, carry):
        for k in range(KV):
            sl = pl.ds(k * L, L)
            base_v[j, sl] = base_v[j, sl] + ttab_v[0, sl]
            base_v[SCHUNK + j, sl] = base_v[SCHUNK + j, sl] + ttab_v[1, sl]
        return carry

    lax.fori_loop(0, SCHUNK, _mkbase, 0)

    for k in range(KV):
        sl = pl.ds(k * L, L)
        gb_v[pl.ds(k * 2 * L, 2 * L)] = plsc.pack(
            g_v[sl], b_v[sl], format=plsc.PackFormat.INTERLEAVED)

    inv_h = jnp.float32(1.0 / H)
    lane0 = jnp.arange(L, dtype=jnp.int32)

    def _start_gather(b, rows_v, gsem):
        idx = ids_v[b, :]
        pltpu.make_async_copy(w_hbm.at[idx], rows_v, gsem).start()

    def _compute(b, rows_v, ob_v):
        tv = ttv[b, :]

        def _ja(j, c2):
            tsp = jnp.take_along_axis(tv, jnp.full((L,), j, jnp.int32),
                                      axis=0)
            r = j + tsp[0] * SCHUNK
            acc_s = [jnp.zeros((L,), jnp.float32) for _ in range(4)]
            acc_q = [jnp.zeros((L,), jnp.float32) for _ in range(4)]
            for k in range(KV):
                sl = pl.ds(k * L, L)
                x = rows_v[j, sl] + base_v[r, sl]
                xbuf_v[j, sl] = x
                acc_s[k % 4] = acc_s[k % 4] + x
                acc_q[k % 4] = acc_q[k % 4] + x * x
            accs_v[j, :] = (acc_s[0] + acc_s[1]) + (acc_s[2] + acc_s[3])
            accq_v[j, :] = (acc_q[0] + acc_q[1]) + (acc_q[2] + acc_q[3])
            return c2

        lax.fori_loop(0, SCHUNK, _ja, 0)

        s4 = [jnp.zeros((L,), jnp.float32) for _ in range(4)]
        q4 = [jnp.zeros((L,), jnp.float32) for _ in range(4)]
        for l in range(L):
            cl = jnp.full((L,), l, jnp.int32)
            s4[l % 4] = s4[l % 4] + plsc.load_gather(accs_v, [lane0, cl])
            q4[l % 4] = q4[l % 4] + plsc.load_gather(accq_v, [lane0, cl])
        s_tot = (s4[0] + s4[1]) + (s4[2] + s4[3])
        q_tot = (q4[0] + q4[1]) + (q4[2] + q4[3])
        mean = s_tot * inv_h
        var = q_tot * inv_h - mean * mean
        vs = var + EPS
        iv = lax.bitcast_convert_type(vs, jnp.int32)
        y = lax.bitcast_convert_type(
            jnp.full((L,), 0x5F3759DF, jnp.int32) - (iv >> 1), jnp.float32)
        for _ in range(3):
            y = y * (1.5 - 0.5 * vs * y * y)
        cvec = mean * y
        ispl = [jnp.take_along_axis(y, jnp.full((L,), j, jnp.int32), axis=0)
                for j in range(SCHUNK)]
        cspl = [jnp.take_along_axis(cvec, jnp.full((L,), j, jnp.int32),
                                    axis=0)
                for j in range(SCHUNK)]

        def _kc(kt, c2):
            for k8 in range(8):
                slk = pl.ds(kt * 128 + k8 * L, L)
                gk, bk = plsc.unpack(gb_v[pl.ds(kt * 256 + k8 * 2 * L, 2 * L)],
                                     format=plsc.PackFormat.INTERLEAVED)
                for j in range(SCHUNK):
                    off = (j // 8) * (6 * 1024) + (j % 8) * 128 + k8 * L
                    t = xbuf_v[j, slk] * ispl[j] - cspl[j]
                    ob_v[pl.ds(kt * 1024 + off, L)] = t * gk + bk
            return c2

        lax.fori_loop(0, H // 128, _kc, 0)

    idx0 = ids_v[0, :]

    def _phase(b, i, rows_v, gsem, ob_v, osem):
        pltpu.make_async_copy(w_hbm.at[idx0], rows_v, gsem).wait()

        @pl.when(i > 0)
        def _():
            pltpu.make_async_copy(ob_v,
                                  out_hbm.at[b, pl.ds(wid * SCHUNK * H,
                                                      SCHUNK * H)],
                                  osem).wait()

        _compute(b, rows_v, ob_v)
        pltpu.make_async_copy(ob_v,
                              out_hbm.at[b, pl.ds(wid * SCHUNK * H,
                                                  SCHUNK * H)],
                              osem).start()
        _start_gather(jnp.minimum(b + 2, B - 1), rows_v, gsem)

    _start_gather(0, rows_a, gsem_a)
    _start_gather(1, rows_b, gsem_b)

    def _pair(i, carry):
        b0 = 2 * i
        _phase(b0, i, rows_a, gsem_a, ob_a, osem_a)
        _phase(b0 + 1, i, rows_b, gsem_b, ob_b, osem_b)
        return carry

    lax.fori_loop(0, B // 2, _pair, 0)
    pltpu.make_async_copy(ob_a,
                          out_hbm.at[B - 2, pl.ds(wid * SCHUNK * H,
                                                  SCHUNK * H)],
                          osem_a).wait()
    pltpu.make_async_copy(ob_b,
                          out_hbm.at[B - 1, pl.ds(wid * SCHUNK * H,
                                                  SCHUNK * H)],
                          osem_b).wait()
    pltpu.make_async_copy(w_hbm.at[idx0], rows_a, gsem_a).wait()
    pltpu.make_async_copy(w_hbm.at[idx0], rows_b, gsem_b).wait()


def kernel(input_ids, token_type_ids, word_embeddings, position_embeddings,
           token_type_embeddings, ln_weight, ln_bias):
    out2 = _emb_ln_kernel(input_ids.astype(jnp.int32),
                          token_type_ids.astype(jnp.int32),
                          word_embeddings, position_embeddings,
                          token_type_embeddings, ln_weight, ln_bias)
    out5 = out2.reshape(B, S // 8, H // 128, 8, 128)
    return out5.transpose(0, 1, 3, 2, 4).reshape(B, S, H)


# speedup vs baseline: 1.0147x; 1.0140x over previous
import functools

import jax
import jax.numpy as jnp
from jax import lax
from jax.experimental import pallas as pl
from jax.experimental.pallas import tpu as pltpu
from jax.experimental.pallas import tpu_sc as plsc

VOCAB = 21128
H = 768
MAX_POS = 512
B = 128
S = 512
EPS = 1e-12

L = 16
NW = 32
SCHUNK = S // NW
KV = H // L
WTOK = B * SCHUNK

_mesh = plsc.VectorSubcoreMesh(core_axis_name="c", subcore_axis_name="s")


@functools.partial(
    pl.kernel,
    out_type=jax.ShapeDtypeStruct((B, S * H), jnp.float32),
    mesh=_mesh,
    compiler_params=pltpu.CompilerParams(use_tc_tiling_on_sc=False,
                                         needs_layout_passes=False),
    scratch_types=[
        pltpu.VMEM((B, SCHUNK), jnp.int32),
        pltpu.VMEM((B, SCHUNK), jnp.int32),
        pltpu.VMEM((2 * SCHUNK, H), jnp.float32),
        pltpu.VMEM((2, H), jnp.float32),
        pltpu.VMEM((H,), jnp.float32),
        pltpu.VMEM((H,), jnp.float32),
        pltpu.VMEM((SCHUNK, H), jnp.float32),
        pltpu.VMEM((SCHUNK, H), jnp.float32),
        pltpu.VMEM((SCHUNK * H,), jnp.float32),
        pltpu.VMEM((SCHUNK * H,), jnp.float32),
        pltpu.VMEM((SCHUNK, H), jnp.float32),
        pltpu.VMEM((SCHUNK, L), jnp.float32),
        pltpu.VMEM((SCHUNK, L), jnp.float32),
        pltpu.VMEM((2 * H,), jnp.bfloat16),
        pltpu.SemaphoreType.DMA,
        pltpu.SemaphoreType.DMA,
        pltpu.SemaphoreType.DMA,
        pltpu.SemaphoreType.DMA,
    ],
)
def _emb_ln_kernel(ids_hbm, tt_hbm, w_hbm, pos_hbm, ttemb_hbm, g_hbm, bb_hbm,
                   out_hbm, ids_v, ttv, base_v, ttab_v, g_v, b_v, rows_a,
                   rows_b, ob_a, ob_b, xbuf_v, accs_v, accq_v, gb_v,
                   gsem_a, gsem_b, osem_a, osem_b):
    wid = lax.axis_index("s") * 2 + lax.axis_index("c")
    s0 = wid * SCHUNK

    pltpu.sync_copy(ids_hbm.at[:, pl.ds(s0, SCHUNK)], ids_v)
    pltpu.sync_copy(tt_hbm.at[:, pl.ds(s0, SCHUNK)], ttv)
    pltpu.sync_copy(pos_hbm.at[pl.ds(s0, SCHUNK)], base_v.at[pl.ds(0, SCHUNK)])
    pltpu.sync_copy(pos_hbm.at[pl.ds(s0, SCHUNK)],
                    base_v.at[pl.ds(SCHUNK, SCHUNK)])
    pltpu.sync_copy(ttemb_hbm, ttab_v)
    pltpu.sync_copy(g_hbm, g_v)
    pltpu.sync_copy(bb_hbm, b_v)

    def _mkbase(j, carry):
        for k in range(KV):
            sl = pl.ds(k * L, L)
            base_v[j, sl] = base_v[j, sl] + ttab_v[0, sl]
            base_v[SCHUNK + j, sl] = base_v[SCHUNK + j, sl] + ttab_v[1, sl]
        return carry

    lax.fori_loop(0, SCHUNK, _mkbase, 0)

    for k in range(KV):
        sl = pl.ds(k * L, L)
        gb_v[pl.ds(k * 2 * L, 2 * L)] = plsc.pack(
            g_v[sl], b_v[sl], format=plsc.PackFormat.INTERLEAVED)

    inv_h = jnp.float32(1.0 / H)
    lane0 = jnp.arange(L, dtype=jnp.int32)

    def _start_gather(b, rows_v, gsem):
        idx = ids_v[b, :]
        pltpu.make_async_copy(w_hbm.at[idx], rows_v, gsem).start()

    def _compute(b, rows_v, ob_v):
        tv = ttv[b, :]

        def _ja(j, c2):
            tsp = jnp.take_along_axis(tv, jnp.full((L,), j, jnp.int32),
                                      axis=0)
            r = j + tsp[0] * SCHUNK
            acc_s = [jnp.zeros((L,), jnp.float32) for _ in range(4)]
            acc_q = [jnp.zeros((L,), jnp.float32) for _ in range(4)]
            for k in range(KV):
                sl = pl.ds(k * L, L)
                x = rows_v[j, sl] + base_v[r, sl]
                xbuf_v[j, sl] = x
                acc_s[k % 4] = acc_s[k % 4] + x
                acc_q[k % 4] = acc_q[k % 4] + x * x
            accs_v[j, :] = (acc_s[0] + acc_s[1]) + (acc_s[2] + acc_s[3])
            accq_v[j, :] = (acc_q[0] + acc_q[1]) + (acc_q[2] + acc_q[3])
            return c2

        lax.fori_loop(0, SCHUNK, _ja, 0)

        s4 = [jnp.zeros((L,), jnp.float32) for _ in range(4)]
        q4 = [jnp.zeros((L,), jnp.float32) for _ in range(4)]
        for l in range(L):
            cl = jnp.full((L,), l, jnp.int32)
            s4[l % 4] = s4[l % 4] + plsc.load_gather(accs_v, [lane0, cl])
            q4[l % 4] = q4[l % 4] + plsc.load_gather(accq_v, [lane0, cl])
        s_tot = (s4[0] + s4[1]) + (s4[2] + s4[3])
        q_tot = (q4[0] + q4[1]) + (q4[2] + q4[3])
        mean = s_tot * inv_h
        var = q_tot * inv_h - mean * mean
        vs = var + EPS
        iv = lax.bitcast_convert_type(vs, jnp.int32)
        y = lax.bitcast_convert_type(
            jnp.full((L,), 0x5F3759DF, jnp.int32) - (iv >> 1), jnp.float32)
        for _ in range(3):
            y = y * (1.5 - 0.5 * vs * y * y)
        cvec = mean * y
        ispl = [jnp.take_along_axis(y, jnp.full((L,), j, jnp.int32), axis=0)
                for j in range(SCHUNK)]
        cspl = [jnp.take_along_axis(cvec, jnp.full((L,), j, jnp.int32),
                                    axis=0)
                for j in range(SCHUNK)]

        def _kc(k, c2):
            gk, bk = plsc.unpack(gb_v[pl.ds(k * 2 * L, 2 * L)],
                                 format=plsc.PackFormat.INTERLEAVED)
            slk = pl.ds(k * L, L)
            kbase = (k // 8) * 1024 + (k % 8) * L
            for j in range(SCHUNK):
                off = (j // 8) * (6 * 1024) + (j % 8) * 128
                t = xbuf_v[j, slk] * ispl[j] - cspl[j]
                ob_v[pl.ds(kbase + off, L)] = t * gk + bk
            return c2

        lax.fori_loop(0, KV, _kc, 0)

    idx0 = ids_v[0, :]

    def _phase(b, i, rows_v, gsem, ob_v, osem):
        pltpu.make_async_copy(w_hbm.at[idx0], rows_v, gsem).wait()

        @pl.when(i > 0)
        def _():
            pltpu.make_async_copy(ob_v,
                                  out_hbm.at[b, pl.ds(wid * SCHUNK * H,
                                                      SCHUNK * H)],
                                  osem).wait()

        _compute(b, rows_v, ob_v)
        pltpu.make_async_copy(ob_v,
                              out_hbm.at[b, pl.ds(wid * SCHUNK * H,
                                                  SCHUNK * H)],
                              osem).start()
        _start_gather(jnp.minimum(b + 2, B - 1), rows_v, gsem)

    _start_gather(0, rows_a, gsem_a)
    _start_gather(1, rows_b, gsem_b)

    def _pair(i, carry):
        b0 = 2 * i
        _phase(b0, i, rows_a, gsem_a, ob_a, osem_a)
        _phase(b0 + 1, i, rows_b, gsem_b, ob_b, osem_b)
        return carry

    lax.fori_loop(0, B // 2, _pair, 0)
    pltpu.make_async_copy(ob_a,
                          out_hbm.at[B - 2, pl.ds(wid * SCHUNK * H,
                                                  SCHUNK * H)],
                          osem_a).wait()
    pltpu.make_async_copy(ob_b,
                          out_hbm.at[B - 1, pl.ds(wid * SCHUNK * H,
                                                  SCHUNK * H)],
                          osem_b).wait()
    pltpu.make_async_copy(w_hbm.at[idx0], rows_a, gsem_a).wait()
    pltpu.make_async_copy(w_hbm.at[idx0], rows_b, gsem_b).wait()


def kernel(input_ids, token_type_ids, word_embeddings, position_embeddings,
           token_type_embeddings, ln_weight, ln_bias):
    out2 = _emb_ln_kernel(input_ids.astype(jnp.int32),
                          token_type_ids.astype(jnp.int32),
                          word_embeddings, position_embeddings,
                          token_type_embeddings, ln_weight, ln_bias)
    out5 = out2.reshape(B, S // 8, H // 128, 8, 128)
    return out5.transpose(0, 1, 3, 2, 4).reshape(B, S, H)


# speedup vs baseline: 1.5891x; 1.5661x over previous
import functools

import jax
import jax.numpy as jnp
from jax import lax
from jax.experimental import pallas as pl
from jax.experimental.pallas import tpu as pltpu
from jax.experimental.pallas import tpu_sc as plsc

VOCAB = 21128
H = 768
MAX_POS = 512
B = 128
S = 512
EPS = 1e-12

L = 16
NW = 32
SCHUNK = S // NW
KV = H // L
WTOK = B * SCHUNK

_mesh = plsc.VectorSubcoreMesh(core_axis_name="c", subcore_axis_name="s")


@functools.partial(
    pl.kernel,
    out_type=jax.ShapeDtypeStruct((B, S, H), jnp.float32),
    mesh=_mesh,
    compiler_params=pltpu.CompilerParams(use_tc_tiling_on_sc=False,
                                         needs_layout_passes=False),
    scratch_types=[
        pltpu.VMEM((B, SCHUNK), jnp.int32),
        pltpu.VMEM((B, SCHUNK), jnp.int32),
        pltpu.VMEM((2 * SCHUNK, H), jnp.float32),
        pltpu.VMEM((2, H), jnp.float32),
        pltpu.VMEM((H,), jnp.float32),
        pltpu.VMEM((H,), jnp.float32),
        pltpu.VMEM((SCHUNK, H), jnp.float32),
        pltpu.VMEM((SCHUNK, H), jnp.float32),
        pltpu.VMEM((SCHUNK, H), jnp.float32),
        pltpu.VMEM((SCHUNK, H), jnp.float32),
        pltpu.VMEM((SCHUNK, H), jnp.float32),
        pltpu.VMEM((SCHUNK, L), jnp.float32),
        pltpu.VMEM((SCHUNK, L), jnp.float32),
        pltpu.VMEM((2 * H,), jnp.bfloat16),
        pltpu.SemaphoreType.DMA,
        pltpu.SemaphoreType.DMA,
        pltpu.SemaphoreType.DMA,
        pltpu.SemaphoreType.DMA,
    ],
)
def _emb_ln_kernel(ids_hbm, tt_hbm, w_hbm, pos_hbm, ttemb_hbm, g_hbm, bb_hbm,
                   out_hbm, ids_v, ttv, base_v, ttab_v, g_v, b_v, rows_a,
                   rows_b, ob_a, ob_b, xbuf_v, accs_v, accq_v, gb_v,
                   gsem_a, gsem_b, osem_a, osem_b):
    wid = lax.axis_index("s") * 2 + lax.axis_index("c")
    s0 = wid * SCHUNK

    pltpu.sync_copy(ids_hbm.at[:, pl.ds(s0, SCHUNK)], ids_v)
    pltpu.sync_copy(tt_hbm.at[:, pl.ds(s0, SCHUNK)], ttv)
    pltpu.sync_copy(pos_hbm.at[pl.ds(s0, SCHUNK)], base_v.at[pl.ds(0, SCHUNK)])
    pltpu.sync_copy(pos_hbm.at[pl.ds(s0, SCHUNK)],
                    base_v.at[pl.ds(SCHUNK, SCHUNK)])
    pltpu.sync_copy(ttemb_hbm, ttab_v)
    pltpu.sync_copy(g_hbm, g_v)
    pltpu.sync_copy(bb_hbm, b_v)

    def _mkbase(j, carry):
        for k in range(KV):
            sl = pl.ds(k * L, L)
            base_v[j, sl] = base_v[j, sl] + ttab_v[0, sl]
            base_v[SCHUNK + j, sl] = base_v[SCHUNK + j, sl] + ttab_v[1, sl]
        return carry

    lax.fori_loop(0, SCHUNK, _mkbase, 0)

    for k in range(KV):
        sl = pl.ds(k * L, L)
        gb_v[pl.ds(k * 2 * L, 2 * L)] = plsc.pack(
            g_v[sl], b_v[sl], format=plsc.PackFormat.INTERLEAVED)

    inv_h = jnp.float32(1.0 / H)
    lane0 = jnp.arange(L, dtype=jnp.int32)

    def _start_gather(b, rows_v, gsem):
        idx = ids_v[b, :]
        pltpu.make_async_copy(w_hbm.at[idx], rows_v, gsem).start()

    def _compute(b, rows_v, ob_v):
        tv = ttv[b, :]

        def _ja(j, c2):
            tsp = jnp.take_along_axis(tv, jnp.full((L,), j, jnp.int32),
                                      axis=0)
            r = j + tsp[0] * SCHUNK
            acc_s = [jnp.zeros((L,), jnp.float32) for _ in range(4)]
            acc_q = [jnp.zeros((L,), jnp.float32) for _ in range(4)]
            for k in range(KV):
                sl = pl.ds(k * L, L)
                x = rows_v[j, sl] + base_v[r, sl]
                xbuf_v[j, sl] = x
                acc_s[k % 4] = acc_s[k % 4] + x
                acc_q[k % 4] = acc_q[k % 4] + x * x
            accs_v[j, :] = (acc_s[0] + acc_s[1]) + (acc_s[2] + acc_s[3])
            accq_v[j, :] = (acc_q[0] + acc_q[1]) + (acc_q[2] + acc_q[3])
            return c2

        lax.fori_loop(0, SCHUNK, _ja, 0)

        s4 = [jnp.zeros((L,), jnp.float32) for _ in range(4)]
        q4 = [jnp.zeros((L,), jnp.float32) for _ in range(4)]
        for l in range(L):
            cl = jnp.full((L,), l, jnp.int32)
            s4[l % 4] = s4[l % 4] + plsc.load_gather(accs_v, [lane0, cl])
            q4[l % 4] = q4[l % 4] + plsc.load_gather(accq_v, [lane0, cl])
        s_tot = (s4[0] + s4[1]) + (s4[2] + s4[3])
        q_tot = (q4[0] + q4[1]) + (q4[2] + q4[3])
        mean = s_tot * inv_h
        var = q_tot * inv_h - mean * mean
        vs = var + EPS
        iv = lax.bitcast_convert_type(vs, jnp.int32)
        y = lax.bitcast_convert_type(
            jnp.full((L,), 0x5F3759DF, jnp.int32) - (iv >> 1), jnp.float32)
        for _ in range(3):
            y = y * (1.5 - 0.5 * vs * y * y)
        cvec = mean * y
        ispl = [jnp.take_along_axis(y, jnp.full((L,), j, jnp.int32), axis=0)
                for j in range(SCHUNK)]
        cspl = [jnp.take_along_axis(cvec, jnp.full((L,), j, jnp.int32),
                                    axis=0)
                for j in range(SCHUNK)]

        def _kc(k, c2):
            gk, bk = plsc.unpack(gb_v[pl.ds(k * 2 * L, 2 * L)],
                                 format=plsc.PackFormat.INTERLEAVED)
            slk = pl.ds(k * L, L)
            for j in range(SCHUNK):
                t = xbuf_v[j, slk] * ispl[j] - cspl[j]
                ob_v[j, slk] = t * gk + bk
            return c2

        lax.fori_loop(0, KV, _kc, 0)

    idx0 = ids_v[0, :]

    def _phase(b, i, rows_v, gsem, ob_v, osem):
        pltpu.make_async_copy(w_hbm.at[idx0], rows_v, gsem).wait()

        @pl.when(i > 0)
        def _():
            pltpu.make_async_copy(ob_v, out_hbm.at[b, pl.ds(s0, SCHUNK)],
                                  osem).wait()

        _compute(b, rows_v, ob_v)
        pltpu.make_async_copy(ob_v, out_hbm.at[b, pl.ds(s0, SCHUNK)],
                              osem).start()
        _start_gather(jnp.minimum(b + 2, B - 1), rows_v, gsem)

    _start_gather(0, rows_a, gsem_a)
    _start_gather(1, rows_b, gsem_b)

    def _pair(i, carry):
        b0 = 2 * i
        _phase(b0, i, rows_a, gsem_a, ob_a, osem_a)
        _phase(b0 + 1, i, rows_b, gsem_b, ob_b, osem_b)
        return carry

    lax.fori_loop(0, B // 2, _pair, 0)
    pltpu.make_async_copy(ob_a, out_hbm.at[B - 2, pl.ds(s0, SCHUNK)],
                          osem_a).wait()
    pltpu.make_async_copy(ob_b, out_hbm.at[B - 1, pl.ds(s0, SCHUNK)],
                          osem_b).wait()
    pltpu.make_async_copy(w_hbm.at[idx0], rows_a, gsem_a).wait()
    pltpu.make_async_copy(w_hbm.at[idx0], rows_b, gsem_b).wait()


def kernel(input_ids, token_type_ids, word_embeddings, position_embeddings,
           token_type_embeddings, ln_weight, ln_bias):
    return _emb_ln_kernel(input_ids.astype(jnp.int32),
                          token_type_ids.astype(jnp.int32),
                          word_embeddings, position_embeddings,
                          token_type_embeddings, ln_weight, ln_bias)
